# Initial kernel scaffold; baseline (speedup 1.0000x reference)
import functools

import jax
import jax.numpy as jnp
from jax import lax
from jax.experimental import pallas as pl
from jax.experimental.pallas import tpu as pltpu
from jax.experimental.pallas import tpu_sc as plsc

N = 100000
DIN = 128
DH = 16
DOUT = 40
E = 3200000

NC = 2
NS = 16
NW = NC * NS
EPW = E // NW
CHUNK = 80
NCHUNKS = EPW // CHUNK

ZBLK = 2000
NBLK = N // ZBLK
ROWBLK = 2000
GRID = N // ROWBLK

_mesh = plsc.VectorSubcoreMesh(core_axis_name="c", subcore_axis_name="s")


def _node_blocks(s):
    out = []
    for j in range((NBLK + NS - 1) // NS):
        out.append(s + j * NS)
    return out


@functools.partial(
    pl.kernel,
    out_type=jax.ShapeDtypeStruct((NC, N), jnp.float32),
    mesh=_mesh,
    scratch_types=[
        pltpu.VMEM_SHARED((N,), jnp.float32),
        pltpu.VMEM((CHUNK,), jnp.int32),
        pltpu.VMEM((CHUNK,), jnp.float32),
    ],
)
def _deg_call(dst_hbm, ew_hbm, zrow_hbm, out_hbm, acc, dstv, ewv):
    c = lax.axis_index("c")
    s = lax.axis_index("s")
    wid = s * NC + c

    for b in _node_blocks(s):
        @pl.when(b < NBLK)
        def _():
            pltpu.sync_copy(zrow_hbm, acc.at[pl.ds(b * ZBLK, ZBLK)])
    plsc.subcore_barrier()

    @pl.loop(0, NCHUNKS)
    def _(j):
        base = wid * EPW + j * CHUNK
        pltpu.sync_copy(dst_hbm.at[pl.ds(base, CHUNK)], dstv)
        pltpu.sync_copy(ew_hbm.at[pl.ds(base, CHUNK)], ewv)
        pltpu.sync_copy(ewv, acc.at[dstv], add=True)

    plsc.subcore_barrier()
    for b in _node_blocks(s):
        @pl.when(b < NBLK)
        def _():
            pltpu.sync_copy(acc.at[pl.ds(b * ZBLK, ZBLK)],
                            out_hbm.at[c, pl.ds(b * ZBLK, ZBLK)])


@functools.partial(
    pl.kernel,
    out_type=jax.ShapeDtypeStruct((NC, N, DH), jnp.float32),
    mesh=_mesh,
    scratch_types=[
        pltpu.VMEM_SHARED((N, DH), jnp.float32),
        pltpu.VMEM((CHUNK,), jnp.int32),
        pltpu.VMEM((CHUNK,), jnp.int32),
        pltpu.VMEM((CHUNK,), jnp.float32),
        pltpu.VMEM((CHUNK, DH), jnp.float32),
        pltpu.SemaphoreType.DMA,
    ],
)
def _prop_call(hp_hbm, src_hbm, dst_hbm, ew_hbm, zblk_hbm, out_hbm,
               acc, srcv, dstv, ewv, rows, sem):
    c = lax.axis_index("c")
    s = lax.axis_index("s")
    wid = s * NC + c

    for b in _node_blocks(s):
        @pl.when(b < NBLK)
        def _():
            pltpu.sync_copy(zblk_hbm, acc.at[pl.ds(b * ZBLK, ZBLK)])
    plsc.subcore_barrier()

    @pl.loop(0, NCHUNKS)
    def _(j):
        base = wid * EPW + j * CHUNK
        pltpu.sync_copy(src_hbm.at[pl.ds(base, CHUNK)], srcv)
        pltpu.sync_copy(dst_hbm.at[pl.ds(base, CHUNK)], dstv)
        pltpu.sync_copy(ew_hbm.at[pl.ds(base, CHUNK)], ewv)
        pltpu.async_copy(hp_hbm.at[srcv], rows, sem).wait()
        for t in range(CHUNK):
            rows[t, :] = rows[t, :] * ewv[t]
        pltpu.sync_copy(rows, acc.at[dstv], add=True)

    plsc.subcore_barrier()
    for b in _node_blocks(s):
        @pl.when(b < NBLK)
        def _():
            pltpu.sync_copy(acc.at[pl.ds(b * ZBLK, ZBLK)],
                            out_hbm.at[c, pl.ds(b * ZBLK, ZBLK)])


def _prep_body(degp_ref, x_ref, w1_ref, dinv_ref, h1p_ref):
    deg = degp_ref[0] + degp_ref[1] + 1.0
    dinv = lax.rsqrt(deg)
    dinv_ref[...] = dinv[:, None]
    h = jnp.dot(x_ref[...], w1_ref[...], preferred_element_type=jnp.float32)
    h1p_ref[...] = h * dinv[:, None]


def _prep_call(degp, x, w1):
    return pl.pallas_call(
        _prep_body,
        grid=(GRID,),
        in_specs=[
            pl.BlockSpec((NC, ROWBLK), lambda i: (0, i)),
            pl.BlockSpec((ROWBLK, DIN), lambda i: (i, 0)),
            pl.BlockSpec((DIN, DH), lambda i: (0, 0)),
        ],
        out_specs=[
            pl.BlockSpec((ROWBLK, 1), lambda i: (i, 0)),
            pl.BlockSpec((ROWBLK, DH), lambda i: (i, 0)),
        ],
        out_shape=[
            jax.ShapeDtypeStruct((N, 1), jnp.float32),
            jax.ShapeDtypeStruct((N, DH), jnp.float32),
        ],
    )(degp, x, w1)


def _mid_body(p_ref, h1p_ref, dinv_ref, b1_ref, gp_ref):
    t = p_ref[0] + p_ref[1] + h1p_ref[...]
    dv = dinv_ref[...]
    g = jnp.maximum(t * dv + b1_ref[...], 0.0)
    gp_ref[...] = g * dv


def _mid_call(p, h1p, dinv, b1):
    return pl.pallas_call(
        _mid_body,
        grid=(GRID,),
        in_specs=[
            pl.BlockSpec((NC, ROWBLK, DH), lambda i: (0, i, 0)),
            pl.BlockSpec((ROWBLK, DH), lambda i: (i, 0)),
            pl.BlockSpec((ROWBLK, 1), lambda i: (i, 0)),
            pl.BlockSpec((1, DH), lambda i: (0, 0)),
        ],
        out_specs=pl.BlockSpec((ROWBLK, DH), lambda i: (i, 0)),
        out_shape=jax.ShapeDtypeStruct((N, DH), jnp.float32),
    )(p, h1p, dinv, b1)


def _fin_body(q_ref, gp_ref, dinv_ref, w2_ref, b2_ref, out_ref):
    t = (q_ref[0] + q_ref[1] + gp_ref[...]) * dinv_ref[...]
    out_ref[...] = (
        jnp.dot(t, w2_ref[...], preferred_element_type=jnp.float32)
        + b2_ref[...]
    )


def _fin_call(q, gp, dinv, w2, b2):
    return pl.pallas_call(
        _fin_body,
        grid=(GRID,),
        in_specs=[
            pl.BlockSpec((NC, ROWBLK, DH), lambda i: (0, i, 0)),
            pl.BlockSpec((ROWBLK, DH), lambda i: (i, 0)),
            pl.BlockSpec((ROWBLK, 1), lambda i: (i, 0)),
            pl.BlockSpec((DH, DOUT), lambda i: (0, 0)),
            pl.BlockSpec((1, DOUT), lambda i: (0, 0)),
        ],
        out_specs=pl.BlockSpec((ROWBLK, DOUT), lambda i: (i, 0)),
        out_shape=jax.ShapeDtypeStruct((N, DOUT), jnp.float32),
    )(q, gp, dinv, w2, b2)


def kernel(x, edge_index, edge_weight, W1, b1, W2, b2):
    src = edge_index[0].astype(jnp.int32)
    dst = edge_index[1].astype(jnp.int32)
    ew = edge_weight.astype(jnp.float32)
    zrow = jnp.zeros((ZBLK,), jnp.float32)
    zblk = jnp.zeros((ZBLK, DH), jnp.float32)

    degp = _deg_call(dst, ew, zrow)
    dinv, h1p = _prep_call(degp, x, W1)
    p = _prop_call(h1p, src, dst, ew, zblk)
    gp = _mid_call(p, h1p, dinv, b1.reshape(1, DH))
    q = _prop_call(gp, src, dst, ew, zblk)
    out = _fin_call(q, gp, dinv, W2, b2.reshape(1, DOUT))
    return out


# speedup vs baseline: 78.5548x; 78.5548x over previous
import functools

import jax
import jax.numpy as jnp
from jax import lax
from jax.experimental import pallas as pl
from jax.experimental.pallas import tpu as pltpu
from jax.experimental.pallas import tpu_sc as plsc

N = 100000
DIN = 128
DH = 16
DOUT = 40
E = 3200000

NC = 2
NS = 16
NW = NC * NS
EPW = E // NW
CHUNK = 80
NCHUNKS = EPW // CHUNK

DGC = 125
DNG = NCHUNKS // DGC
PGC = 25
PNG = NCHUNKS // PGC
NBUF = 5
ZBLK = 2000
NBLK = N // ZBLK
ROWBLK = 10000
GRID = N // ROWBLK

def _node_blocks(s):
    out = []
    for j in range((NBLK + NS - 1) // NS):
        out.append(s + j * NS)
    return out


def _deg_kernel(dst_hbm, ew_hbm, zrow_hbm, out_hbm, acc, dstb, ewb, buf,
                ssem, isems):
    c = lax.axis_index("c")
    sid = lax.axis_index("s")
    wid = sid * NC + c
    wbase = wid * NCHUNKS

    pltpu.sync_copy(zrow_hbm, buf)
    for b in _node_blocks(sid):
        @pl.when(b < NBLK)
        def _():
            pltpu.sync_copy(buf, acc.at[pl.ds(b * ZBLK, ZBLK)])
    plsc.subcore_barrier()

    pltpu.async_copy(dst_hbm.at[pl.ds(wbase, DGC)], dstb.at[0], isems[0])
    pltpu.async_copy(ew_hbm.at[pl.ds(wbase, DGC)], ewb.at[0], isems[0])

    def do_group(go2, par):
        g = go2 * 2 + par
        gbase = wbase + g * DGC

        @pl.when(g > 0)
        def _():
            @pl.loop(0, DGC)
            def _(_k):
                pltpu.make_async_copy(
                    ewb.at[1 - par, 0], acc.at[dstb.at[1 - par, 0]], ssem
                ).wait()

        pltpu.make_async_copy(dst_hbm.at[pl.ds(gbase, DGC)], dstb.at[par],
                              isems[par]).wait()
        pltpu.make_async_copy(ew_hbm.at[pl.ds(gbase, DGC)], ewb.at[par],
                              isems[par]).wait()

        @pl.when(g + 1 < DNG)
        def _():
            pltpu.async_copy(dst_hbm.at[pl.ds(gbase + DGC, DGC)],
                             dstb.at[1 - par], isems[1 - par])
            pltpu.async_copy(ew_hbm.at[pl.ds(gbase + DGC, DGC)],
                             ewb.at[1 - par], isems[1 - par])

        @pl.loop(0, DGC)
        def _(k):
            pltpu.async_copy(ewb.at[par, k], acc.at[dstb.at[par, k]], ssem,
                             add=True)

    @pl.loop(0, DNG // 2)
    def _(go2):
        do_group(go2, 0)
        do_group(go2, 1)

    @pl.loop(0, DGC)
    def _(_k):
        pltpu.make_async_copy(ewb.at[1, 0], acc.at[dstb.at[1, 0]], ssem).wait()

    plsc.subcore_barrier()
    for b in _node_blocks(sid):
        @pl.when(b < NBLK)
        def _():
            pltpu.sync_copy(acc.at[pl.ds(b * ZBLK, ZBLK)], buf)
            pltpu.sync_copy(buf, out_hbm.at[pl.ds(c * N + b * ZBLK, ZBLK)])


def _prop_kernel(hp_hbm, src_hbm, dst_hbm, ew_hbm, zblk_hbm, out_hbm,
                 acc, srcb, dstb, ewb, rows_g, rows_s, gsems, ssems, isems):
    c = lax.axis_index("c")
    sid = lax.axis_index("s")
    wid = sid * NC + c
    wbase = wid * NCHUNKS

    for b in _node_blocks(sid):
        @pl.when(b < NBLK)
        def _():
            pltpu.sync_copy(zblk_hbm, acc.at[pl.ds(b * ZBLK, ZBLK)])
    plsc.subcore_barrier()

    pltpu.async_copy(src_hbm.at[pl.ds(wbase, PGC)], srcb.at[0], isems[0])
    pltpu.async_copy(dst_hbm.at[pl.ds(wbase, PGC)], dstb.at[0], isems[0])
    pltpu.async_copy(ew_hbm.at[pl.ds(wbase, PGC)], ewb.at[0], isems[0])

    def do_group(go2, par):
        g = go2 * 2 + par
        gbase = wbase + g * PGC

        @pl.when(g > 0)
        def _():
            for sl in range(NBUF):
                pltpu.make_async_copy(rows_s.at[sl],
                                      acc.at[dstb.at[par, sl]],
                                      ssems[sl]).wait()

        pltpu.make_async_copy(src_hbm.at[pl.ds(gbase, PGC)], srcb.at[par],
                              isems[par]).wait()
        pltpu.make_async_copy(dst_hbm.at[pl.ds(gbase, PGC)], dstb.at[par],
                              isems[par]).wait()
        pltpu.make_async_copy(ew_hbm.at[pl.ds(gbase, PGC)], ewb.at[par],
                              isems[par]).wait()

        @pl.when(g + 1 < PNG)
        def _():
            pltpu.async_copy(src_hbm.at[pl.ds(gbase + PGC, PGC)],
                             srcb.at[1 - par], isems[1 - par])
            pltpu.async_copy(dst_hbm.at[pl.ds(gbase + PGC, PGC)],
                             dstb.at[1 - par], isems[1 - par])
            pltpu.async_copy(ew_hbm.at[pl.ds(gbase + PGC, PGC)],
                             ewb.at[1 - par], isems[1 - par])

        for sl in range(NBUF):
            pltpu.async_copy(hp_hbm.at[srcb.at[par, sl]], rows_g.at[sl],
                             gsems[sl])

        @pl.loop(0, PGC // NBUF)
        def _(ib):
            for sl in range(NBUF):
                k = ib * NBUF + sl
                pltpu.make_async_copy(hp_hbm.at[srcb.at[par, k]],
                                      rows_g.at[sl], gsems[sl]).wait()

                @pl.when(ib > 0)
                def _():
                    pltpu.make_async_copy(rows_s.at[sl],
                                          acc.at[dstb.at[par, k]],
                                          ssems[sl]).wait()

                for q in range(CHUNK // 16):
                    ew16 = ewb[par, k, pl.ds(q * 16, 16)]
                    for t in range(16):
                        i = q * 16 + t
                        rows_s[sl, i, :] = rows_g[sl, i, :] * ew16[t]

                @pl.when(k + NBUF < PGC)
                def _():
                    pltpu.async_copy(hp_hbm.at[srcb.at[par, k + NBUF]],
                                     rows_g.at[sl], gsems[sl])

                pltpu.async_copy(rows_s.at[sl], acc.at[dstb.at[par, k]],
                                 ssems[sl], add=True)

    @pl.loop(0, PNG // 2)
    def _(go2):
        do_group(go2, 0)
        do_group(go2, 1)

    for sl in range(NBUF):
        pltpu.make_async_copy(rows_s.at[sl], acc.at[dstb.at[0, sl]],
                              ssems[sl]).wait()

    plsc.subcore_barrier()
    for b in _node_blocks(sid):
        @pl.when(b < NBLK)
        def _():
            pltpu.sync_copy(acc.at[pl.ds(b * ZBLK, ZBLK)],
                            out_hbm.at[pl.ds(c * N + b * ZBLK, ZBLK)])


def _prep_body(degp_ref, x_ref, w1_ref, dinv_ref, h1p_ref):
    deg = degp_ref[:, 0] + degp_ref[:, 1] + 1.0
    dinv = lax.rsqrt(deg)
    dinv_ref[...] = dinv[:, None]
    h = jnp.dot(x_ref[...], w1_ref[...], preferred_element_type=jnp.float32)
    h1p_ref[...] = h * dinv[:, None]


def _prep_call(degp, x, w1):
    return pl.pallas_call(
        _prep_body,
        grid=(GRID,),
        in_specs=[
            pl.BlockSpec((ROWBLK, NC), lambda i: (i, 0)),
            pl.BlockSpec((ROWBLK, DIN), lambda i: (i, 0)),
            pl.BlockSpec((DIN, DH), lambda i: (0, 0)),
        ],
        out_specs=[
            pl.BlockSpec((ROWBLK, 1), lambda i: (i, 0)),
            pl.BlockSpec((ROWBLK, DH), lambda i: (i, 0)),
        ],
        out_shape=[
            jax.ShapeDtypeStruct((N, 1), jnp.float32),
            jax.ShapeDtypeStruct((N, DH), jnp.float32),
        ],
    )(degp, x, w1)


def _mid_body(p_ref, h1p_ref, dinv_ref, b1_ref, gp_ref):
    t = p_ref[0] + p_ref[1] + h1p_ref[...]
    dv = dinv_ref[...]
    g = jnp.maximum(t * dv + b1_ref[...], 0.0)
    gp_ref[...] = g * dv


def _mid_call(p, h1p, dinv, b1):
    return pl.pallas_call(
        _mid_body,
        grid=(GRID,),
        in_specs=[
            pl.BlockSpec((NC, ROWBLK, DH), lambda i: (0, i, 0)),
            pl.BlockSpec((ROWBLK, DH), lambda i: (i, 0)),
            pl.BlockSpec((ROWBLK, 1), lambda i: (i, 0)),
            pl.BlockSpec((1, DH), lambda i: (0, 0)),
        ],
        out_specs=pl.BlockSpec((ROWBLK, DH), lambda i: (i, 0)),
        out_shape=jax.ShapeDtypeStruct((N, DH), jnp.float32),
    )(p, h1p, dinv, b1)


def _fin_body(q_ref, gp_ref, dinv_ref, w2_ref, b2_ref, out_ref):
    t = (q_ref[0] + q_ref[1] + gp_ref[...]) * dinv_ref[...]
    out_ref[...] = (
        jnp.dot(t, w2_ref[...], preferred_element_type=jnp.float32)
        + b2_ref[...]
    )


def _fin_call(q, gp, dinv, w2, b2):
    return pl.pallas_call(
        _fin_body,
        grid=(GRID,),
        in_specs=[
            pl.BlockSpec((NC, ROWBLK, DH), lambda i: (0, i, 0)),
            pl.BlockSpec((ROWBLK, DH), lambda i: (i, 0)),
            pl.BlockSpec((ROWBLK, 1), lambda i: (i, 0)),
            pl.BlockSpec((DH, DOUT), lambda i: (0, 0)),
            pl.BlockSpec((1, DOUT), lambda i: (0, 0)),
        ],
        out_specs=pl.BlockSpec((ROWBLK, DOUT), lambda i: (i, 0)),
        out_shape=jax.ShapeDtypeStruct((N, DOUT), jnp.float32),
    )(q, gp, dinv, w2, b2)


@functools.cache
def _sc_calls():
    mesh = plsc.VectorSubcoreMesh(
        core_axis_name="c", subcore_axis_name="s", num_cores=NC, num_subcores=NS
    )
    deg = pl.kernel(
        _deg_kernel,
        out_type=jax.ShapeDtypeStruct((NC * N,), jnp.float32),
        mesh=mesh,
        scratch_types=[
            pltpu.VMEM_SHARED((N,), jnp.float32),
            pltpu.VMEM((2, DGC, CHUNK), jnp.int32),
            pltpu.VMEM((2, DGC, CHUNK), jnp.float32),
            pltpu.VMEM((ZBLK,), jnp.float32),
            pltpu.SemaphoreType.DMA,
            [pltpu.SemaphoreType.DMA] * 2,
        ],
        compiler_params=pltpu.CompilerParams(use_tc_tiling_on_sc=False),
    )
    prop = pl.kernel(
        _prop_kernel,
        out_type=jax.ShapeDtypeStruct((NC * N, DH), jnp.float32),
        mesh=mesh,
        scratch_types=[
            pltpu.VMEM_SHARED((N, DH), jnp.float32),
            pltpu.VMEM((2, PGC, CHUNK), jnp.int32),
            pltpu.VMEM((2, PGC, CHUNK), jnp.int32),
            pltpu.VMEM((2, PGC, CHUNK), jnp.float32),
            pltpu.VMEM((NBUF, CHUNK, DH), jnp.float32),
            pltpu.VMEM((NBUF, CHUNK, DH), jnp.float32),
            [pltpu.SemaphoreType.DMA] * NBUF,
            [pltpu.SemaphoreType.DMA] * NBUF,
            [pltpu.SemaphoreType.DMA] * 2,
        ],
        compiler_params=pltpu.CompilerParams(use_tc_tiling_on_sc=False),
    )
    return deg, prop


def kernel(x, edge_index, edge_weight, W1, b1, W2, b2):
    _deg_call, _prop_call = _sc_calls()
    src = edge_index[0].astype(jnp.int32).reshape(E // CHUNK, CHUNK)
    dst = edge_index[1].astype(jnp.int32).reshape(E // CHUNK, CHUNK)
    ew = edge_weight.astype(jnp.float32).reshape(E // CHUNK, CHUNK)
    zrow = jnp.zeros((ZBLK,), jnp.float32)
    zblk = jnp.zeros((ZBLK, DH), jnp.float32)

    degp = _deg_call(dst, ew, zrow)
    degp = degp.reshape(NC, N).T
    dinv, h1p = _prep_call(degp, x, W1)
    p = _prop_call(h1p, src, dst, ew, zblk).reshape(NC, N, DH)
    gp = _mid_call(p, h1p, dinv, b1.reshape(1, DH))
    q = _prop_call(gp, src, dst, ew, zblk).reshape(NC, N, DH)
    out = _fin_call(q, gp, dinv, W2, b2.reshape(1, DOUT))
    return out
